# Initial kernel scaffold; baseline (speedup 1.0000x reference)
import jax
import jax.numpy as jnp
from jax.experimental import pallas as pl


def kernel(hidden_states, router_weight, w1, w2, shared_gate_up, shared_down):
    raise NotImplementedError("write your pallas kernel here")


# speedup vs baseline: 1.5623x; 1.5623x over previous
import functools

import jax
import jax.numpy as jnp
from jax import lax
from jax.experimental import pallas as pl
from jax.experimental.pallas import tpu as pltpu
from jax.experimental.pallas import tpu_sc as plsc

E = 8
TOPK = 2
D = 2048
N = 1664
NS = 3328
T = 2048
A = T * TOPK
TM = 256
NT = (A + E * TM) // TM
APAD = NT * TM
TN = N // 2
TNS = NS // 4

_DN11 = (((1,), (1,)), ((), ()))


def _bf(x):
    return x.astype(jnp.bfloat16)


def _router_body(x_ref, rw_ref, pos_ref, sc_ref, meta_ref):
    x = _bf(x_ref[...])
    rw = _bf(rw_ref[...])
    logits = lax.dot_general(x, rw, _DN11, preferred_element_type=jnp.float32)

    iota8 = lax.broadcasted_iota(jnp.int32, (T, E), 1)
    m0 = jnp.max(logits, axis=1, keepdims=True)
    i0 = jnp.min(jnp.where(logits == m0, iota8, E), axis=1, keepdims=True)
    sel0 = iota8 == i0
    l1 = jnp.where(sel0, -jnp.inf, logits)
    m1 = jnp.max(l1, axis=1, keepdims=True)
    i1 = jnp.min(jnp.where(l1 == m1, iota8, E), axis=1, keepdims=True)
    sel1 = iota8 == i1
    e1 = jnp.exp(m1 - m0)
    den = 1.0 + e1
    s0 = 1.0 / den
    s1 = e1 / den

    Hc = (sel0.astype(jnp.int32) + sel1.astype(jnp.int32))
    k = 1
    while k < T:
        Hc = Hc + jnp.concatenate(
            [jnp.zeros((k, E), jnp.int32), Hc[: T - k, :]], axis=0)
        k *= 2
    counts = Hc[T - 1 : T, :]
    ntile = (counts + TM - 1) // TM
    padded = ntile * TM

    p = padded
    k = 1
    while k < E:
        p = p + jnp.concatenate(
            [jnp.zeros((1, k), jnp.int32), p[:, : E - k]], axis=1)
        k *= 2
    off = p - padded

    off_b = jnp.broadcast_to(off, (T, E))
    r0 = jnp.sum(jnp.where(sel0, Hc, 0), axis=1, keepdims=True) - 1
    r1 = jnp.sum(jnp.where(sel1, Hc, 0), axis=1, keepdims=True) - 1
    o0 = jnp.sum(jnp.where(sel0, off_b, 0), axis=1, keepdims=True)
    o1 = jnp.sum(jnp.where(sel1, off_b, 0), axis=1, keepdims=True)
    d0 = o0 + r0
    d1 = o1 + r1

    lane2 = lax.broadcasted_iota(jnp.int32, (T, 2), 1)
    pos_ref[...] = jnp.where(lane2 == 0, jnp.broadcast_to(d0, (T, 2)),
                             jnp.broadcast_to(d1, (T, 2)))
    sc_ref[...] = jnp.where(lane2 == 0, jnp.broadcast_to(s0, (T, 2)),
                            jnp.broadcast_to(s1, (T, 2)))

    cum_incl = (off + padded) // TM
    gg = lax.broadcasted_iota(jnp.int32, (32, E), 0)
    cum_b = jnp.broadcast_to(cum_incl, (32, E))
    te = jnp.sum((cum_b <= gg).astype(jnp.int32), axis=1, keepdims=True)
    total = jnp.broadcast_to(cum_incl[:, E - 1 : E], (32, 1))
    g1 = lax.broadcasted_iota(jnp.int32, (32, 1), 0)
    tv = (g1 < total).astype(jnp.int32)
    lane8 = lax.broadcasted_iota(jnp.int32, (32, E), 1)
    meta_ref[...] = (jnp.where(lane8 == 0, jnp.broadcast_to(te, (32, E)), 0)
                     + jnp.where(lane8 == 1, jnp.broadcast_to(tv, (32, E)), 0))


def _router(x, rw):
    return pl.pallas_call(
        _router_body,
        out_shape=(
            jax.ShapeDtypeStruct((T, 2), jnp.int32),
            jax.ShapeDtypeStruct((T, 2), jnp.float32),
            jax.ShapeDtypeStruct((32, E), jnp.int32),
        ),
    )(x, rw)


def _dispatch(x, pos2d, tok2d):
    info = plsc.get_sparse_core_info()
    NC, NSUB = info.num_cores, info.num_subcores
    NW = NC * NSUB
    per_w = A // NW
    n_ch = per_w // 16

    mesh = plsc.VectorSubcoreMesh(core_axis_name="c", subcore_axis_name="s")

    @functools.partial(
        pl.kernel,
        out_type=jax.ShapeDtypeStruct((APAD, D), jnp.float32),
        mesh=mesh,
        scratch_types=[
            pltpu.VMEM((n_ch, 16), jnp.int32),
            pltpu.VMEM((n_ch, 16), jnp.int32),
            pltpu.VMEM((16, D), jnp.float32),
            pltpu.SemaphoreType.DMA,
        ],
    )
    def body(x_hbm, pos_hbm, tok_hbm, xg_hbm, idx_v, tok_v, rows_v, sem):
        wid = lax.axis_index("s") * NC + lax.axis_index("c")
        pltpu.sync_copy(pos_hbm.at[pl.ds(wid * n_ch, n_ch)], idx_v)
        pltpu.sync_copy(tok_hbm.at[pl.ds(wid * n_ch, n_ch)], tok_v)
        for c in range(n_ch):
            pltpu.async_copy(x_hbm.at[tok_v.at[c]], rows_v, sem).wait()
            pltpu.async_copy(rows_v, xg_hbm.at[idx_v.at[c]], sem).wait()

    return body(x, pos2d, tok2d)


def _gmm1_body(te_ref, tv_ref, x_ref, wg_ref, wu_ref, o_ref):
    m = pl.program_id(1)

    @pl.when(tv_ref[m] != 0)
    def _():
        x = _bf(x_ref[...])
        g = lax.dot_general(x, _bf(wg_ref[0]), _DN11,
                            preferred_element_type=jnp.float32)
        u = lax.dot_general(x, _bf(wu_ref[0]), _DN11,
                            preferred_element_type=jnp.float32)
        o_ref[0] = (g * jax.nn.sigmoid(g)) * u


def _gmm1(xg, w1, te, tv):
    nj = N // TN
    grid = (nj, NT)
    return pl.pallas_call(
        _gmm1_body,
        grid_spec=pltpu.PrefetchScalarGridSpec(
            num_scalar_prefetch=2,
            grid=grid,
            in_specs=[
                pl.BlockSpec((TM, D), lambda j, m, te, tv: (m, 0)),
                pl.BlockSpec((1, TN, D), lambda j, m, te, tv: (te[m], j, 0)),
                pl.BlockSpec((1, TN, D), lambda j, m, te, tv: (te[m], nj + j, 0)),
            ],
            out_specs=pl.BlockSpec((1, TM, TN), lambda j, m, te, tv: (j, m, 0)),
        ),
        out_shape=jax.ShapeDtypeStruct((nj, APAD, TN), jnp.float32),
    )(te, tv, xg, w1, w1)


def _gmm2_body(te_ref, tv_ref, i0_ref, i1_ref, w2_ref, o_ref):
    m = pl.program_id(0)

    @pl.when(tv_ref[m] != 0)
    def _():
        ii = jnp.concatenate([i0_ref[0], i1_ref[0]], axis=1)
        o_ref[...] = lax.dot_general(_bf(ii), _bf(w2_ref[0]), _DN11,
                                     preferred_element_type=jnp.float32)


def _gmm2(inter, w2, te, tv):
    return pl.pallas_call(
        _gmm2_body,
        grid_spec=pltpu.PrefetchScalarGridSpec(
            num_scalar_prefetch=2,
            grid=(NT,),
            in_specs=[
                pl.BlockSpec((1, TM, TN), lambda m, te, tv: (0, m, 0)),
                pl.BlockSpec((1, TM, TN), lambda m, te, tv: (1, m, 0)),
                pl.BlockSpec((1, D, N), lambda m, te, tv: (te[m], 0, 0)),
            ],
            out_specs=pl.BlockSpec((TM, D), lambda m, te, tv: (m, 0)),
        ),
        out_shape=jax.ShapeDtypeStruct((APAD, D), jnp.float32),
    )(te, tv, inter, inter, w2)


def _combine(yg, pos2d, sc2d):
    info = plsc.get_sparse_core_info()
    NC, NSUB = info.num_cores, info.num_subcores
    NW = NC * NSUB
    per_w = T // NW
    n_ch = per_w // 8

    mesh = plsc.VectorSubcoreMesh(core_axis_name="c", subcore_axis_name="s")

    @functools.partial(
        pl.kernel,
        out_type=jax.ShapeDtypeStruct((T, D), jnp.float32),
        mesh=mesh,
        scratch_types=[
            pltpu.VMEM((n_ch, 16), jnp.int32),
            pltpu.VMEM((n_ch, 16), jnp.float32),
            pltpu.VMEM((16, D), jnp.float32),
            pltpu.VMEM((8, D), jnp.float32),
            pltpu.SemaphoreType.DMA,
        ],
    )
    def body(yg_hbm, pos_hbm, sc_hbm, out_hbm, idx_v, sc_v, rows_v, ob_v, sem):
        wid = lax.axis_index("s") * NC + lax.axis_index("c")
        pltpu.sync_copy(pos_hbm.at[pl.ds(wid * n_ch, n_ch)], idx_v)
        pltpu.sync_copy(sc_hbm.at[pl.ds(wid * n_ch, n_ch)], sc_v)
        for c in range(n_ch):
            pltpu.async_copy(yg_hbm.at[idx_v.at[c]], rows_v, sem).wait()
            srow = sc_v[c]
            s = [srow[t] for t in range(16)]

            def dbody(dd, _):
                sl = pl.ds(dd * 16, 16)
                for i in range(8):
                    ob_v[i, sl] = (rows_v[2 * i, sl] * s[2 * i]
                                   + rows_v[2 * i + 1, sl] * s[2 * i + 1])
                return 0

            lax.fori_loop(0, D // 16, dbody, 0)
            pltpu.sync_copy(ob_v, out_hbm.at[pl.ds(wid * per_w + c * 8, 8)])

    return body(yg, pos2d, sc2d)


def _shared1_body(x_ref, wg_ref, wu_ref, o_ref):
    x = _bf(x_ref[...])
    g = lax.dot_general(x, _bf(wg_ref[...]), _DN11,
                        preferred_element_type=jnp.float32)
    u = lax.dot_general(x, _bf(wu_ref[...]), _DN11,
                        preferred_element_type=jnp.float32)
    o_ref[0] = (g * jax.nn.sigmoid(g)) * u


def _shared1(x, sgu):
    njs = NS // TNS
    return pl.pallas_call(
        _shared1_body,
        grid=(njs, T // TM),
        in_specs=[
            pl.BlockSpec((TM, D), lambda j, m: (m, 0)),
            pl.BlockSpec((TNS, D), lambda j, m: (j, 0)),
            pl.BlockSpec((TNS, D), lambda j, m: (njs + j, 0)),
        ],
        out_specs=pl.BlockSpec((1, TM, TNS), lambda j, m: (j, m, 0)),
        out_shape=jax.ShapeDtypeStruct((njs, T, TNS), jnp.float32),
    )(x, sgu, sgu)


def _shared2_body(i0_ref, i1_ref, i2_ref, i3_ref, wd_ref, sp_ref, o_ref):
    ii = jnp.concatenate(
        [i0_ref[0], i1_ref[0], i2_ref[0], i3_ref[0]], axis=1)
    y = lax.dot_general(_bf(ii), _bf(wd_ref[...]), _DN11,
                        preferred_element_type=jnp.float32)
    o_ref[...] = y + sp_ref[...]


def _shared2(inter_s, sd, sparse):
    DC = 1024
    njs = NS // TNS
    ispecs = [
        pl.BlockSpec((1, TM, TNS), lambda jd, m, jj=jj: (jj, m, 0))
        for jj in range(njs)
    ]
    return pl.pallas_call(
        _shared2_body,
        grid=(D // DC, T // TM),
        in_specs=ispecs + [
            pl.BlockSpec((DC, NS), lambda jd, m: (jd, 0)),
            pl.BlockSpec((TM, DC), lambda jd, m: (m, jd)),
        ],
        out_specs=pl.BlockSpec((TM, DC), lambda jd, m: (m, jd)),
        out_shape=jax.ShapeDtypeStruct((T, D), jnp.float32),
    )(inter_s, inter_s, inter_s, inter_s, sd, sparse)


def kernel(hidden_states, router_weight, w1, w2, shared_gate_up, shared_down):
    b, s_, d_ = hidden_states.shape
    x = hidden_states.reshape(b * s_, d_)

    pos, scores, meta = _router(x, router_weight)
    te = jnp.minimum(meta[:NT, 0], E - 1)
    tv = meta[:NT, 1]
    pos2d = pos.reshape(A // 16, 16)
    sc2d = scores.reshape(A // 16, 16)

    tok2d = (jnp.arange(A, dtype=jnp.int32) // TOPK).reshape(A // 16, 16)
    xg = _dispatch(x, pos2d, tok2d)
    inter = _gmm1(xg, w1, te, tv)
    yg = _gmm2(inter, w2, te, tv)
    sparse = _combine(yg, pos2d, sc2d)

    inter_s = _shared1(x, shared_gate_up)
    out = _shared2(inter_s, shared_down, sparse)
    return out.reshape(b, s_, d_)
